# speedup vs baseline: 1.0106x; 1.0106x over previous
import functools

import jax
import jax.numpy as jnp
from jax.experimental import pallas as pl
from jax.experimental.pallas import tpu as pltpu

N = 10000
E = 320000
NC = 1000
H = 64

_ROW_BLK = 2000


def _mlp_body(x_ref, w1_ref, b1_ref, g_ref, be_ref, w2_ref, b2_ref, out_ref):
    h = jnp.dot(x_ref[...], w1_ref[...], preferred_element_type=jnp.float32)
    h = h + b1_ref[...]
    mu = jnp.mean(h, axis=-1, keepdims=True)
    var = jnp.mean((h - mu) ** 2, axis=-1, keepdims=True)
    h = (h - mu) * jax.lax.rsqrt(var + 1e-5) * g_ref[...] + be_ref[...]
    h = jnp.maximum(h, 0.0)
    out_ref[...] = (
        jnp.dot(h, w2_ref[...], preferred_element_type=jnp.float32) + b2_ref[...]
    )


def _mlp(x, W1, b1, g, be, W2, b2):
    n, c = x.shape
    grid = n // _ROW_BLK
    return pl.pallas_call(
        _mlp_body,
        grid=(grid,),
        in_specs=[
            pl.BlockSpec((_ROW_BLK, c), lambda i: (i, 0)),
            pl.BlockSpec((c, H), lambda i: (0, 0)),
            pl.BlockSpec((1, H), lambda i: (0, 0)),
            pl.BlockSpec((1, H), lambda i: (0, 0)),
            pl.BlockSpec((1, H), lambda i: (0, 0)),
            pl.BlockSpec((H, c), lambda i: (0, 0)),
            pl.BlockSpec((1, c), lambda i: (0, 0)),
        ],
        out_specs=pl.BlockSpec((_ROW_BLK, c), lambda i: (i, 0)),
        out_shape=jax.ShapeDtypeStruct((n, c), jnp.float32),
    )(x, W1, b1.reshape(1, H), g.reshape(1, H), be.reshape(1, H), W2,
      b2.reshape(1, c))


def kernel(x, edge_index, cluster,
           W1_0, b1_0, g_0, be_0, W2_0, b2_0,
           W1_1, b1_1, g_1, be_1, W2_1, b2_1,
           W1_2, b1_2, g_2, be_2, W2_2, b2_2):
    src = edge_index[0]
    dst = edge_index[1]
    params = [
        (W1_0, b1_0, g_0, be_0, W2_0, b2_0),
        (W1_1, b1_1, g_1, be_1, W2_1, b2_1),
        (W1_2, b1_2, g_2, be_2, W2_2, b2_2),
    ]
    for p in params:
        x2 = _mlp(x, *p)
        agg = jax.ops.segment_max(x2[src], dst, num_segments=N)
        agg = jnp.where(jnp.isneginf(agg), 0.0, agg)
        x = jnp.concatenate([x2, agg], axis=1)
    pooled = jax.ops.segment_max(x, cluster, num_segments=NC)
    pooled = jnp.where(jnp.isneginf(pooled), 0.0, pooled)
    norm = jnp.linalg.norm(pooled, axis=0) + 1e-6
    return pooled / norm
